# baseline (speedup 1.0000x reference)
import functools

import jax
import jax.numpy as jnp
from jax import lax
from jax.experimental import pallas as pl
from jax.experimental.pallas import tpu as pltpu
from jax.experimental.pallas import tpu_sc as plsc

V = 1000000
D = 64
B = 16384
F = 26
N = B * F

_info = plsc.get_sparse_core_info()
NC, NS, L = _info.num_cores, _info.num_subcores, _info.num_lanes
NW = NC * NS
BPW = B // NW
CB = 128
NCB = BPW // CB
NCH = F * NCB


def _softplus(x):
    t = jnp.exp(-jnp.abs(x))
    z = t / (t + 2.0)
    z2 = z * z
    return jnp.maximum(x, 0.0) + z * (
        2.0
        + z2 * (2.0 / 3.0 + z2 * (2.0 / 5.0 + z2 * (2.0 / 7.0 + z2 * (2.0 / 9.0))))
    )


def _body(idx_hbm, center, offset, lo, hi, idx_v, c2, o2, lo_t, hi_t,
          gsem0, gsem1, osem0, osem1):
    wid = lax.axis_index("s") * NC + lax.axis_index("c")
    wb = wid * BPW
    pltpu.sync_copy(idx_hbm.at[:, pl.ds(wb, BPW)], idx_v)

    gsems = (gsem0, gsem1)
    osems = (osem0, osem1)

    def fire_gather(f, c, b):
        ids = idx_v.at[f, pl.ds(c * CB, CB)]
        pltpu.async_copy(center.at[ids], c2.at[b], gsems[b])
        pltpu.async_copy(offset.at[ids], o2.at[b], gsems[b])

    def wait_gather(b):
        ids = idx_v.at[0, pl.ds(0, CB)]
        pltpu.make_async_copy(center.at[ids], c2.at[b], gsems[b]).wait()
        pltpu.make_async_copy(offset.at[ids], o2.at[b], gsems[b]).wait()

    def fire_out(f, c, b):
        tc = wid * NCB + c
        for tr in range(D // 8):
            sl = pl.ds(tr * 8, 8)
            pltpu.async_copy(lo_t.at[b, sl], lo.at[f, tr, tc], osems[b])
            pltpu.async_copy(hi_t.at[b, sl], hi.at[f, tr, tc], osems[b])

    def wait_out(b):
        for tr in range(D // 8):
            sl = pl.ds(tr * 8, 8)
            pltpu.make_async_copy(lo_t.at[b, sl], lo.at[0, tr, 0], osems[b]).wait()
            pltpu.make_async_copy(hi_t.at[b, sl], hi.at[0, tr, 0], osems[b]).wait()

    def compute(b):
        def row(r, carry):
            col = jnp.full((L,), 0, jnp.int32) + r
            for s in range(D // L):
                sl = pl.ds(s * L, L)
                rows = jax.lax.iota(jnp.int32, L) + (s * L)
                c = c2[b, r, sl]
                sp = _softplus(o2[b, r, sl])
                plsc.store_scatter(lo_t.at[b], [rows, col], c - sp)
                plsc.store_scatter(hi_t.at[b], [rows, col], c + sp)
            return carry

        lax.fori_loop(0, CB, row, 0)

    fire_gather(0, 0, 0)

    def step(j2, carry):
        for ph in range(2):
            j = j2 * 2 + ph
            f = j // NCB
            c = j % NCB
            jn = j + 1

            @pl.when(jn < NCH)
            def _():
                fire_gather(jn // NCB, jn % NCB, 1 - ph)

            wait_gather(ph)

            @pl.when(j >= 2)
            def _():
                wait_out(ph)

            compute(ph)
            fire_out(f, c, ph)
        return carry

    lax.fori_loop(0, NCH // 2, step, 0)
    wait_out(0)
    wait_out(1)


def _run(idx_t, center, offset):
    mesh = plsc.VectorSubcoreMesh(core_axis_name="c", subcore_axis_name="s")
    f = functools.partial(
        pl.kernel,
        mesh=mesh,
        out_type=[
            jax.ShapeDtypeStruct((F, D // 8, B // 128, 8, 128), jnp.float32),
            jax.ShapeDtypeStruct((F, D // 8, B // 128, 8, 128), jnp.float32),
        ],
        scratch_types=[
            pltpu.VMEM((F, BPW), jnp.int32),
            pltpu.VMEM((2, CB, D), jnp.float32),
            pltpu.VMEM((2, CB, D), jnp.float32),
            pltpu.VMEM((2, D, CB), jnp.float32),
            pltpu.VMEM((2, D, CB), jnp.float32),
            pltpu.SemaphoreType.DMA,
            pltpu.SemaphoreType.DMA,
            pltpu.SemaphoreType.DMA,
            pltpu.SemaphoreType.DMA,
        ],
        compiler_params=pltpu.CompilerParams(
            use_tc_tiling_on_sc=False, needs_layout_passes=False
        ),
    )(_body)
    return f(idx_t, center, offset)


def kernel(idx, center, offset):
    idx_t = jnp.swapaxes(idx.astype(jnp.int32), 0, 1)
    lo5, hi5 = _run(idx_t, center, offset)
    def _to_bfd(x):
        return jnp.transpose(x, (2, 4, 0, 1, 3)).reshape(B, F, D)

    return (_to_bfd(lo5), _to_bfd(hi5))


# speedup vs baseline: 2.0241x; 2.0241x over previous
import functools

import jax
import jax.numpy as jnp
from jax import lax
from jax.experimental import pallas as pl
from jax.experimental.pallas import tpu as pltpu
from jax.experimental.pallas import tpu_sc as plsc

V = 1000000
D = 64
B = 16384
F = 26
N = B * F

_info = plsc.get_sparse_core_info()
NC, NS, L = _info.num_cores, _info.num_subcores, _info.num_lanes
NW = NC * NS
BPW = B // NW
CHB = 4
CH = CHB * F
NCH = BPW // CHB


def _softplus(x):
    t = jnp.exp(-jnp.abs(x))
    z = t / (t + 2.0)
    z2 = z * z
    return jnp.maximum(x, 0.0) + z * (
        2.0
        + z2 * (2.0 / 3.0 + z2 * (2.0 / 5.0 + z2 * (2.0 / 7.0 + z2 * (2.0 / 9.0))))
    )


def _body(idx_hbm, center, offset, lo, hi, idx_s, idx_v, c2, o2, lo3, hi3,
          gsem0, gsem1, osem0, osem1):
    wid = lax.axis_index("s") * NC + lax.axis_index("c")
    wb = wid * BPW
    pltpu.sync_copy(idx_hbm.at[pl.ds(wb, BPW)], idx_s)

    def repack(q, carry):
        for p in range(CHB):
            r = q * CHB + p
            idx_v[q, pl.ds(F * p, L)] = idx_s[r, pl.ds(0, L)]
            idx_v[q, pl.ds(F * p + F - L, L)] = idx_s[r, pl.ds(F - L, L)]
        return carry

    lax.fori_loop(0, NCH, repack, 0)

    gsems = (gsem0, gsem1)
    osems = (osem0, osem1)

    def fire_gather(j, b):
        pltpu.async_copy(center.at[idx_v.at[j]], c2.at[b], gsems[b])
        pltpu.async_copy(offset.at[idx_v.at[j]], o2.at[b], gsems[b])

    def wait_gather(b):
        pltpu.make_async_copy(center.at[idx_v.at[0]], c2.at[b], gsems[b]).wait()
        pltpu.make_async_copy(offset.at[idx_v.at[0]], o2.at[b], gsems[b]).wait()

    def fire_out(j, b):
        dst = pl.ds(wb + j * CHB, CHB)
        pltpu.async_copy(lo3.at[b], lo.at[dst], osems[b])
        pltpu.async_copy(hi3.at[b], hi.at[dst], osems[b])

    def wait_out(b):
        dst = pl.ds(wb, CHB)
        pltpu.make_async_copy(lo3.at[b], lo.at[dst], osems[b]).wait()
        pltpu.make_async_copy(hi3.at[b], hi.at[dst], osems[b]).wait()

    def compute(b):
        for bb in range(CHB):
            def row(f, carry):
                r = bb * F + f
                for s in range(D // L):
                    sl = pl.ds(s * L, L)
                    c = c2[b, r, sl]
                    sp = _softplus(o2[b, r, sl])
                    lo3[b, bb, f, sl] = c - sp
                    hi3[b, bb, f, sl] = c + sp
                return carry

            lax.fori_loop(0, F, row, 0)

    fire_gather(0, 0)

    def step(j2, carry):
        for ph in range(2):
            j = j2 * 2 + ph

            @pl.when(j + 1 < NCH)
            def _():
                fire_gather(j + 1, 1 - ph)

            wait_gather(ph)

            @pl.when(j >= 2)
            def _():
                wait_out(ph)

            compute(ph)
            fire_out(j, ph)
        return carry

    lax.fori_loop(0, NCH // 2, step, 0)
    wait_out(0)
    wait_out(1)


def _run(idx, center, offset):
    mesh = plsc.VectorSubcoreMesh(core_axis_name="c", subcore_axis_name="s")
    f = functools.partial(
        pl.kernel,
        mesh=mesh,
        out_type=[
            jax.ShapeDtypeStruct((B, F, D), jnp.float32),
            jax.ShapeDtypeStruct((B, F, D), jnp.float32),
        ],
        scratch_types=[
            pltpu.VMEM((BPW, F), jnp.int32),
            pltpu.VMEM((NCH, CH), jnp.int32),
            pltpu.VMEM((2, CH, D), jnp.float32),
            pltpu.VMEM((2, CH, D), jnp.float32),
            pltpu.VMEM((2, CHB, F, D), jnp.float32),
            pltpu.VMEM((2, CHB, F, D), jnp.float32),
            pltpu.SemaphoreType.DMA,
            pltpu.SemaphoreType.DMA,
            pltpu.SemaphoreType.DMA,
            pltpu.SemaphoreType.DMA,
        ],
        compiler_params=pltpu.CompilerParams(use_tc_tiling_on_sc=False),
    )(_body)
    return f(idx, center, offset)


def kernel(idx, center, offset):
    lo, hi = _run(idx.astype(jnp.int32), center, offset)
    return (lo, hi)
